# baseline (speedup 1.0000x reference)
import functools

import numpy as np

import jax
import jax.numpy as jnp
from jax.experimental import pallas as pl
from jax.experimental.pallas import tpu as pltpu

BT = 256

_PERM_W1 = np.array(
    [(r % 32) * 3 + (r // 32) if r < 96 else r for r in range(128)], np.int32)


def _fused_kernel(x_ref, w1_ref, b1_ref, w2_ref, b2_ref,
                  wf1_ref, bf1_ref, wf2_ref, bf2_ref, wf3_ref, bf3_ref,
                  out_ref):
    f32, bf16 = jnp.float32, jnp.bfloat16
    B = out_ref.shape[0]

    xt = jnp.transpose(x_ref[...], (1, 0, 2))
    xg = jnp.stack([xt[:, :, g * 32:(g + 1) * 32] for g in range(4)],
                   axis=1)
    x96 = xg.reshape(96, B, 32)
    zc = jnp.zeros((1, B, 32), bf16)
    ch = [x96[ci * 32:(ci + 1) * 32] for ci in range(3)]
    chs = [jnp.concatenate([c[1:], zc], axis=0) for c in ch]
    zpad = jnp.zeros((32, B, 32), bf16)
    xv = jnp.concatenate(ch + [zpad] + chs + [zpad], axis=2)
    xv = xv.reshape(32 * B, 256)

    acc1 = jnp.dot(xv[0:28 * B], w1_ref[0], preferred_element_type=f32)
    for p in range(1, 3):
        acc1 = acc1 + jnp.dot(xv[2 * p * B:(2 * p + 28) * B], w1_ref[p],
                              preferred_element_type=f32)
    y1 = jnp.maximum(acc1 + b1_ref[...], 0.0)

    c1 = jnp.maximum(y1[:, 0:128], y1[:, 128:256]).astype(bf16)
    c1r = c1.reshape(14, 2 * B, 128)
    p1 = jnp.maximum(c1r[:, :B, :], c1r[:, B:, :]).reshape(14 * B, 128)

    p1s = jnp.concatenate([p1[B:], p1[:B]], axis=0)
    x2 = jnp.concatenate([p1, p1s], axis=1)
    acc2 = jnp.dot(x2[0:10 * B], w2_ref[0], preferred_element_type=f32)
    for p in range(1, 3):
        acc2 = acc2 + jnp.dot(x2[2 * p * B:(2 * p + 10) * B], w2_ref[p],
                              preferred_element_type=f32)
    y2 = jnp.maximum(acc2 + b2_ref[...], 0.0)

    c2 = jnp.maximum(y2[:, 0:128], y2[:, 128:256]).astype(bf16)
    c2r = c2.reshape(5, 2 * B, 128)
    p2 = jnp.maximum(c2r[:, :B, :], c2r[:, B:, :]).reshape(5 * B, 128)

    accf = jnp.dot(p2[0:B], wf1_ref[0], preferred_element_type=f32)
    for h in range(1, 5):
        accf = accf + jnp.dot(p2[h * B:(h + 1) * B], wf1_ref[h],
                              preferred_element_type=f32)
    a1 = jnp.maximum(accf + bf1_ref[...], 0.0)

    a2 = jnp.maximum(jnp.dot(a1.astype(bf16), wf2_ref[...],
                             preferred_element_type=f32) + bf2_ref[...], 0.0)
    a3 = jnp.dot(a2.astype(bf16), wf3_ref[...],
                 preferred_element_type=f32) + bf3_ref[...]
    out_ref[...] = a3[:, 0:16]


def kernel(x_nchw, w1, b1, s1e, s1o, w2, b2, s2e, s2o,
           wf1, bf1, wf2, bf2, wf3, bf3):
    del s1e, s1o, s2e, s2o
    n, c, h, w = x_nchw.shape
    assert (c, h, w) == (3, 32, 32)
    n_pad = ((n + BT - 1) // BT) * BT

    perm256 = jnp.asarray(np.concatenate([_PERM_W1, _PERM_W1 + 128]))
    w1p = jnp.take(w1, perm256, axis=1)

    xr = x_nchw.astype(jnp.bfloat16).reshape(n, 24, 128)
    if n_pad != n:
        xr = jnp.pad(xr, ((0, n_pad - n), (0, 0), (0, 0)))

    flops = int(n_pad * 2 * (28 * 28 * 75 * 6 + 10 * 10 * 150 * 16
                             + 400 * 120 + 120 * 84 + 84 * 10) * 1.3)
    bytes_accessed = int(xr.size * 2 + 2 * 1024 * 1024 + n_pad * 128 * 4)

    out = pl.pallas_call(
        _fused_kernel,
        out_shape=jax.ShapeDtypeStruct((n_pad, 16), jnp.float32),
        grid=(n_pad // BT,),
        in_specs=[
            pl.BlockSpec((BT, 24, 128), lambda i: (i, 0, 0)),
            pl.BlockSpec((3, 256, 256), lambda i: (0, 0, 0)),
            pl.BlockSpec((1, 256), lambda i: (0, 0)),
            pl.BlockSpec((3, 256, 256), lambda i: (0, 0, 0)),
            pl.BlockSpec((1, 256), lambda i: (0, 0)),
            pl.BlockSpec((5, 128, 128), lambda i: (0, 0, 0)),
            pl.BlockSpec((1, 128), lambda i: (0, 0)),
            pl.BlockSpec((128, 128), lambda i: (0, 0)),
            pl.BlockSpec((1, 128), lambda i: (0, 0)),
            pl.BlockSpec((128, 128), lambda i: (0, 0)),
            pl.BlockSpec((1, 128), lambda i: (0, 0)),
        ],
        out_specs=pl.BlockSpec((BT, 16), lambda i: (i, 0)),
        compiler_params=pltpu.CompilerParams(
            dimension_semantics=("arbitrary",),
            vmem_limit_bytes=64 * 1024 * 1024,
        ),
        cost_estimate=pl.CostEstimate(flops=flops, transcendentals=0,
                                      bytes_accessed=bytes_accessed),
    )(xr, w1p, b1, w2, b2, wf1, bf1, wf2, bf2, wf3, bf3)

    return out[:n, :10]


# speedup vs baseline: 1.0213x; 1.0213x over previous
import functools

import numpy as np

import jax
import jax.numpy as jnp
from jax.experimental import pallas as pl
from jax.experimental.pallas import tpu as pltpu

BT = 256

_PERM_W1 = np.array(
    [(r % 32) * 3 + (r // 32) if r < 96 else r for r in range(128)], np.int32)


def _forward_chunk(xc, w1_ref, b1_ref, w2_ref, b2_ref,
                   wf1_ref, bf1_ref, wf2_ref, bf2_ref, wf3_ref, bf3_ref):
    f32, bf16 = jnp.float32, jnp.bfloat16
    B = xc.shape[0]

    x96 = jnp.transpose(xc, (1, 0, 2))
    zc = jnp.zeros((1, B, 32), bf16)
    ch = [x96[ci * 32:(ci + 1) * 32] for ci in range(3)]
    chs = [jnp.concatenate([c[1:], zc], axis=0) for c in ch]
    zpad = jnp.zeros((32, B, 32), bf16)
    xv = jnp.concatenate(ch + [zpad] + chs + [zpad], axis=2)
    xv = xv.reshape(32 * B, 256)

    acc1 = jnp.dot(xv[0:28 * B], w1_ref[0], preferred_element_type=f32)
    for p in range(1, 3):
        acc1 = acc1 + jnp.dot(xv[2 * p * B:(2 * p + 28) * B], w1_ref[p],
                              preferred_element_type=f32)

    c1 = jnp.maximum(acc1[:, 0:128], acc1[:, 128:256])
    c1r = c1.reshape(14, 2 * B, 128)
    p1f = jnp.maximum(c1r[:, :B, :], c1r[:, B:, :])
    p1 = jnp.maximum(p1f + b1_ref[...], 0.0).astype(bf16).reshape(14 * B, 128)

    p1s = jnp.concatenate([p1[B:], p1[:B]], axis=0)
    x2 = jnp.concatenate([p1, p1s], axis=1)
    acc2 = jnp.dot(x2[0:10 * B], w2_ref[0], preferred_element_type=f32)
    for p in range(1, 3):
        acc2 = acc2 + jnp.dot(x2[2 * p * B:(2 * p + 10) * B], w2_ref[p],
                              preferred_element_type=f32)
    c2 = jnp.maximum(acc2[:, 0:128], acc2[:, 128:256])
    c2r = c2.reshape(5, 2 * B, 128)
    p2f = jnp.maximum(c2r[:, :B, :], c2r[:, B:, :])
    p2 = jnp.maximum(p2f + b2_ref[...], 0.0).astype(bf16).reshape(5 * B, 128)

    accf = jnp.dot(p2[0:B], wf1_ref[0], preferred_element_type=f32)
    for h in range(1, 5):
        accf = accf + jnp.dot(p2[h * B:(h + 1) * B], wf1_ref[h],
                              preferred_element_type=f32)
    a1 = jnp.maximum(accf + bf1_ref[...], 0.0)

    a2 = jnp.maximum(jnp.dot(a1.astype(bf16), wf2_ref[...],
                             preferred_element_type=f32) + bf2_ref[...], 0.0)
    a3 = jnp.dot(a2.astype(bf16), wf3_ref[...],
                 preferred_element_type=f32) + bf3_ref[...]
    return a3[:, 0:16]


def _fused_kernel(x_ref, w1_ref, b1_ref, w2_ref, b2_ref,
                  wf1_ref, bf1_ref, wf2_ref, bf2_ref, wf3_ref, bf3_ref,
                  out_ref):
    out_ref[...] = _forward_chunk(
        x_ref[...], w1_ref, b1_ref, w2_ref, b2_ref,
        wf1_ref, bf1_ref, wf2_ref, bf2_ref, wf3_ref, bf3_ref)


def kernel(x_nchw, w1, b1, s1e, s1o, w2, b2, s2e, s2o,
           wf1, bf1, wf2, bf2, wf3, bf3):
    del s1e, s1o, s2e, s2o
    n, c, h, w = x_nchw.shape
    assert (c, h, w) == (3, 32, 32)
    n_pad = ((n + BT - 1) // BT) * BT

    perm256 = jnp.asarray(np.concatenate([_PERM_W1, _PERM_W1 + 128]))
    w1p = jnp.take(w1, perm256, axis=1)
    b1h = b1[:, 0:128]
    b2h = b2[:, 0:128]

    xr = x_nchw.astype(jnp.bfloat16).reshape(n, 96, 32)
    if n_pad != n:
        xr = jnp.pad(xr, ((0, n_pad - n), (0, 0), (0, 0)))

    flops = int(n_pad * 2 * (28 * 28 * 75 * 6 + 10 * 10 * 150 * 16
                             + 400 * 120 + 120 * 84 + 84 * 10) * 1.3)
    bytes_accessed = int(xr.size * 2 + 2 * 1024 * 1024 + n_pad * 128 * 4)

    out = pl.pallas_call(
        _fused_kernel,
        out_shape=jax.ShapeDtypeStruct((n_pad, 16), jnp.float32),
        grid=(n_pad // BT,),
        in_specs=[
            pl.BlockSpec((BT, 96, 32), lambda i: (i, 0, 0)),
            pl.BlockSpec((3, 256, 256), lambda i: (0, 0, 0)),
            pl.BlockSpec((1, 128), lambda i: (0, 0)),
            pl.BlockSpec((3, 256, 256), lambda i: (0, 0, 0)),
            pl.BlockSpec((1, 128), lambda i: (0, 0)),
            pl.BlockSpec((5, 128, 128), lambda i: (0, 0, 0)),
            pl.BlockSpec((1, 128), lambda i: (0, 0)),
            pl.BlockSpec((128, 128), lambda i: (0, 0)),
            pl.BlockSpec((1, 128), lambda i: (0, 0)),
            pl.BlockSpec((128, 128), lambda i: (0, 0)),
            pl.BlockSpec((1, 128), lambda i: (0, 0)),
        ],
        out_specs=pl.BlockSpec((BT, 16), lambda i: (i, 0)),
        compiler_params=pltpu.CompilerParams(
            dimension_semantics=("arbitrary",),
            vmem_limit_bytes=64 * 1024 * 1024,
        ),
        cost_estimate=pl.CostEstimate(flops=flops, transcendentals=0,
                                      bytes_accessed=bytes_accessed),
    )(xr, w1p, b1h, w2, b2h, wf1, bf1, wf2, bf2, wf3, bf3)

    return out[:n, :10]


# speedup vs baseline: 1.0218x; 1.0005x over previous
import functools

import numpy as np

import jax
import jax.numpy as jnp
from jax.experimental import pallas as pl
from jax.experimental.pallas import tpu as pltpu

BT = 512

_PERM_W1 = np.array(
    [(r % 32) * 3 + (r // 32) if r < 96 else r for r in range(128)], np.int32)


def _forward_chunk(xc, w1_ref, b1_ref, w2_ref, b2_ref,
                   wf1_ref, bf1_ref, wf2_ref, bf2_ref, wf3_ref, bf3_ref):
    f32, bf16 = jnp.float32, jnp.bfloat16
    B = xc.shape[0]

    x96 = jnp.transpose(xc, (1, 0, 2))
    zc = jnp.zeros((1, B, 32), bf16)
    ch = [x96[ci * 32:(ci + 1) * 32] for ci in range(3)]
    chs = [jnp.concatenate([c[1:], zc], axis=0) for c in ch]
    zpad = jnp.zeros((32, B, 32), bf16)
    xv = jnp.concatenate(ch + [zpad] + chs + [zpad], axis=2)
    xv = xv.reshape(32 * B, 256)

    acc1 = jnp.dot(xv[0:28 * B], w1_ref[0], preferred_element_type=f32)
    for p in range(1, 3):
        acc1 = acc1 + jnp.dot(xv[2 * p * B:(2 * p + 28) * B], w1_ref[p],
                              preferred_element_type=f32)

    c1 = jnp.maximum(acc1[:, 0:128], acc1[:, 128:256])
    c1r = c1.reshape(14, 2 * B, 128)
    p1f = jnp.maximum(c1r[:, :B, :], c1r[:, B:, :])
    p1 = jnp.maximum(p1f + b1_ref[...], 0.0).astype(bf16).reshape(14 * B, 128)

    p1s = jnp.concatenate([p1[B:], p1[:B]], axis=0)
    x2 = jnp.concatenate([p1, p1s], axis=1)
    acc2 = jnp.dot(x2[0:10 * B], w2_ref[0], preferred_element_type=f32)
    for p in range(1, 3):
        acc2 = acc2 + jnp.dot(x2[2 * p * B:(2 * p + 10) * B], w2_ref[p],
                              preferred_element_type=f32)
    c2 = jnp.maximum(acc2[:, 0:128], acc2[:, 128:256])
    c2r = c2.reshape(5, 2 * B, 128)
    p2f = jnp.maximum(c2r[:, :B, :], c2r[:, B:, :])
    p2 = jnp.maximum(p2f + b2_ref[...], 0.0).astype(bf16).reshape(5 * B, 128)

    accf = jnp.dot(p2[0:B], wf1_ref[0], preferred_element_type=f32)
    for h in range(1, 5):
        accf = accf + jnp.dot(p2[h * B:(h + 1) * B], wf1_ref[h],
                              preferred_element_type=f32)
    a1 = jnp.maximum(accf + bf1_ref[...], 0.0)

    a2 = jnp.maximum(jnp.dot(a1.astype(bf16), wf2_ref[...],
                             preferred_element_type=f32) + bf2_ref[...], 0.0)
    a3 = jnp.dot(a2.astype(bf16), wf3_ref[...],
                 preferred_element_type=f32) + bf3_ref[...]
    return a3[:, 0:16]


def _fused_kernel(x_ref, w1_ref, b1_ref, w2_ref, b2_ref,
                  wf1_ref, bf1_ref, wf2_ref, bf2_ref, wf3_ref, bf3_ref,
                  out_ref):
    out_ref[...] = _forward_chunk(
        x_ref[...], w1_ref, b1_ref, w2_ref, b2_ref,
        wf1_ref, bf1_ref, wf2_ref, bf2_ref, wf3_ref, bf3_ref)


def kernel(x_nchw, w1, b1, s1e, s1o, w2, b2, s2e, s2o,
           wf1, bf1, wf2, bf2, wf3, bf3):
    del s1e, s1o, s2e, s2o
    n, c, h, w = x_nchw.shape
    assert (c, h, w) == (3, 32, 32)
    n_pad = ((n + BT - 1) // BT) * BT

    perm256 = jnp.asarray(np.concatenate([_PERM_W1, _PERM_W1 + 128]))
    w1p = jnp.take(w1, perm256, axis=1)
    b1h = b1[:, 0:128]
    b2h = b2[:, 0:128]

    xr = x_nchw.astype(jnp.bfloat16).reshape(n, 96, 32)
    if n_pad != n:
        xr = jnp.pad(xr, ((0, n_pad - n), (0, 0), (0, 0)))

    flops = int(n_pad * 2 * (28 * 28 * 75 * 6 + 10 * 10 * 150 * 16
                             + 400 * 120 + 120 * 84 + 84 * 10) * 1.3)
    bytes_accessed = int(xr.size * 2 + 2 * 1024 * 1024 + n_pad * 128 * 4)

    out = pl.pallas_call(
        _fused_kernel,
        out_shape=jax.ShapeDtypeStruct((n_pad, 16), jnp.float32),
        grid=(n_pad // BT,),
        in_specs=[
            pl.BlockSpec((BT, 96, 32), lambda i: (i, 0, 0)),
            pl.BlockSpec((3, 256, 256), lambda i: (0, 0, 0)),
            pl.BlockSpec((1, 128), lambda i: (0, 0)),
            pl.BlockSpec((3, 256, 256), lambda i: (0, 0, 0)),
            pl.BlockSpec((1, 128), lambda i: (0, 0)),
            pl.BlockSpec((5, 128, 128), lambda i: (0, 0, 0)),
            pl.BlockSpec((1, 128), lambda i: (0, 0)),
            pl.BlockSpec((128, 128), lambda i: (0, 0)),
            pl.BlockSpec((1, 128), lambda i: (0, 0)),
            pl.BlockSpec((128, 128), lambda i: (0, 0)),
            pl.BlockSpec((1, 128), lambda i: (0, 0)),
        ],
        out_specs=pl.BlockSpec((BT, 16), lambda i: (i, 0)),
        compiler_params=pltpu.CompilerParams(
            dimension_semantics=("arbitrary",),
            vmem_limit_bytes=64 * 1024 * 1024,
        ),
        cost_estimate=pl.CostEstimate(flops=flops, transcendentals=0,
                                      bytes_accessed=bytes_accessed),
    )(xr, w1p, b1h, w2, b2h, wf1, bf1, wf2, bf2, wf3, bf3)

    return out[:n, :10]
